# baseline (speedup 1.0000x reference)
import functools

import jax
import jax.numpy as jnp
from jax.experimental import pallas as pl
from jax.experimental.pallas import tpu as pltpu


_LOG2E = 1.4426950408889634


def _attn_kernel(self_ref, feats_ref, neigh_ref, a_ref, out_ref, vt_ref):
    d = self_ref.shape[1]

    @pl.when(pl.program_id(0) == 0)
    def _():
        a2 = a_ref[d:, :]
        vt_ref[...] = (feats_ref[...] @ (a2 * _LOG2E)).T

    a1 = a_ref[:d, :]
    u = self_ref[...] @ (a1 * _LOG2E)
    t = u + vt_ref[...]
    t = jnp.maximum(t, 0.2 * t)
    t = jnp.where(neigh_ref[...] != 0, t, -jnp.inf)
    p = jnp.exp2(t)
    l = jnp.sum(p, axis=1, keepdims=True)
    o = jnp.dot(p.astype(jnp.bfloat16), feats_ref[...].astype(jnp.bfloat16),
                preferred_element_type=jnp.float32)
    out_ref[...] = o * (1.0 / jnp.where(l == 0.0, 1.0, l))


@jax.jit
def kernel(self_feats, features_neighs, neigh_matrix, a):
    n, d = self_feats.shape
    m = features_neighs.shape[0]
    bn = 1024
    grid = (n // bn,)
    return pl.pallas_call(
        _attn_kernel,
        grid=grid,
        in_specs=[
            pl.BlockSpec((bn, d), lambda i: (i, 0)),
            pl.BlockSpec((m, d), lambda i: (0, 0)),
            pl.BlockSpec((bn, m), lambda i: (i, 0)),
            pl.BlockSpec((2 * d, 1), lambda i: (0, 0)),
        ],
        out_specs=pl.BlockSpec((bn, d), lambda i: (i, 0)),
        out_shape=jax.ShapeDtypeStruct((n, d), jnp.float32),
        scratch_shapes=[pltpu.VMEM((1, m), jnp.float32)],
        compiler_params=pltpu.CompilerParams(
            dimension_semantics=("arbitrary",),
        ),
    )(self_feats, features_neighs, neigh_matrix, a)


# speedup vs baseline: 1.0124x; 1.0124x over previous
import functools

import jax
import jax.numpy as jnp
from jax.experimental import pallas as pl
from jax.experimental.pallas import tpu as pltpu


_LOG2E = 1.4426950408889634


def _attn_kernel(self_ref, feats_ref, neigh_ref, a_ref, out_ref, vt_ref, fb_ref):
    d = self_ref.shape[1]

    @pl.when(pl.program_id(0) == 0)
    def _():
        a2 = a_ref[d:, :]
        vt_ref[...] = (feats_ref[...] @ (a2 * _LOG2E)).T
        fb_ref[...] = feats_ref[...].astype(jnp.bfloat16)

    a1 = a_ref[:d, :]
    u = self_ref[...] @ (a1 * _LOG2E)
    t = u + vt_ref[...]
    t = jnp.maximum(t, 0.2 * t)
    t = jnp.where(neigh_ref[...] != 0, t, -jnp.inf)
    p = jnp.exp2(t)
    l = jnp.sum(p, axis=1, keepdims=True)
    o = jnp.dot(p.astype(jnp.bfloat16), fb_ref[...],
                preferred_element_type=jnp.float32)
    out_ref[...] = o * (1.0 / jnp.where(l == 0.0, 1.0, l))


@jax.jit
def kernel(self_feats, features_neighs, neigh_matrix, a):
    n, d = self_feats.shape
    m = features_neighs.shape[0]
    bn = 512
    grid = (n // bn,)
    return pl.pallas_call(
        _attn_kernel,
        grid=grid,
        in_specs=[
            pl.BlockSpec((bn, d), lambda i: (i, 0)),
            pl.BlockSpec((m, d), lambda i: (0, 0)),
            pl.BlockSpec((bn, m), lambda i: (i, 0)),
            pl.BlockSpec((2 * d, 1), lambda i: (0, 0)),
        ],
        out_specs=pl.BlockSpec((bn, d), lambda i: (i, 0)),
        out_shape=jax.ShapeDtypeStruct((n, d), jnp.float32),
        scratch_shapes=[pltpu.VMEM((1, m), jnp.float32),
                        pltpu.VMEM((m, d), jnp.bfloat16)],
        compiler_params=pltpu.CompilerParams(
            dimension_semantics=("arbitrary",),
        ),
    )(self_feats, features_neighs, neigh_matrix, a)


# speedup vs baseline: 1.1098x; 1.0962x over previous
import functools

import jax
import jax.numpy as jnp
from jax.experimental import pallas as pl
from jax.experimental.pallas import tpu as pltpu


_LOG2E = 1.4426950408889634


def _attn_kernel(self_ref, feats_ref, neigh_ref, a_ref, out_ref, vt_ref, fb_ref):
    d = self_ref.shape[1]

    @pl.when(pl.program_id(0) == 0)
    def _():
        a2 = a_ref[d:, :]
        vt_ref[...] = (feats_ref[...] @ (a2 * _LOG2E)).T
        m = feats_ref.shape[0]
        col = jax.lax.broadcasted_iota(jnp.int32, (m, 128), 1)
        fb_ref[:, :128] = jnp.where(col == 0, 1.0, 0.0).astype(jnp.bfloat16)
        fb_ref[:, 128:] = feats_ref[...].astype(jnp.bfloat16)

    a1 = a_ref[:d, :]
    u = self_ref[...] @ (a1 * _LOG2E)
    t = u + vt_ref[...]
    t = jnp.maximum(t, 0.2 * t)
    t = jnp.where(neigh_ref[...] != 0, t, -jnp.inf)
    p = jnp.exp2(t).astype(jnp.bfloat16)
    o = jnp.dot(p, fb_ref[...],
                preferred_element_type=jnp.float32)
    l = o[:, 0:1]
    out_ref[...] = o[:, 128:] * (1.0 / jnp.where(l == 0.0, 1.0, l))


@jax.jit
def kernel(self_feats, features_neighs, neigh_matrix, a):
    n, d = self_feats.shape
    m = features_neighs.shape[0]
    bn = 512
    grid = (n // bn,)
    return pl.pallas_call(
        _attn_kernel,
        grid=grid,
        in_specs=[
            pl.BlockSpec((bn, d), lambda i: (i, 0)),
            pl.BlockSpec((m, d), lambda i: (0, 0)),
            pl.BlockSpec((bn, m), lambda i: (i, 0)),
            pl.BlockSpec((2 * d, 1), lambda i: (0, 0)),
        ],
        out_specs=pl.BlockSpec((bn, d), lambda i: (i, 0)),
        out_shape=jax.ShapeDtypeStruct((n, d), jnp.float32),
        scratch_shapes=[pltpu.VMEM((1, m), jnp.float32),
                        pltpu.VMEM((m, 128 + d), jnp.bfloat16)],
        compiler_params=pltpu.CompilerParams(
            dimension_semantics=("arbitrary",),
        ),
    )(self_feats, features_neighs, neigh_matrix, a)
